# baseline (speedup 1.0000x reference)
import jax
import jax.numpy as jnp
from jax.experimental import pallas as pl
from jax.experimental.pallas import tpu as pltpu

_CHUNK = 1000
_NBUF = 4


def _affine_kernel(x_hbm, w_ref, b_ref, o_hbm, xbuf, obuf, ld_sem, st_sem):
    n = x_hbm.shape[0]
    nc = n // _CHUNK
    w = w_ref[...]
    bias = b_ref[...]

    def ld(i, slot):
        return pltpu.make_async_copy(
            x_hbm.at[pl.ds(i * _CHUNK, _CHUNK), :], xbuf.at[slot],
            ld_sem.at[slot])

    def st(i, slot):
        return pltpu.make_async_copy(
            obuf.at[slot], o_hbm.at[pl.ds(i * _CHUNK, _CHUNK), :],
            st_sem.at[slot])

    for i in range(min(_NBUF, nc)):
        ld(i, i).start()
    for i in range(nc):
        slot = i % _NBUF
        ld(i, slot).wait()
        if i >= _NBUF:
            st(i - _NBUF, slot).wait()
        obuf[slot] = (
            jnp.dot(xbuf[slot], w, preferred_element_type=jnp.float32) + bias
        )
        st(i, slot).start()
        if i + _NBUF < nc:
            ld(i + _NBUF, slot).start()
    for i in range(max(nc - _NBUF, 0), nc):
        st(i, i % _NBUF).wait()


def kernel(nodes, edges, senders, receivers, W, b):
    n, d = nodes.shape
    b2 = b.reshape(1, d)
    return pl.pallas_call(
        _affine_kernel,
        in_specs=[
            pl.BlockSpec(memory_space=pltpu.MemorySpace.HBM),
            pl.BlockSpec(memory_space=pltpu.VMEM),
            pl.BlockSpec(memory_space=pltpu.VMEM),
        ],
        out_specs=pl.BlockSpec(memory_space=pltpu.MemorySpace.HBM),
        out_shape=jax.ShapeDtypeStruct((n, d), jnp.float32),
        scratch_shapes=[
            pltpu.VMEM((_NBUF, _CHUNK, d), jnp.float32),
            pltpu.VMEM((_NBUF, _CHUNK, d), jnp.float32),
            pltpu.SemaphoreType.DMA((_NBUF,)),
            pltpu.SemaphoreType.DMA((_NBUF,)),
        ],
    )(nodes, W, b2)


# speedup vs baseline: 1.0259x; 1.0259x over previous
import jax
import jax.numpy as jnp
from jax.experimental import pallas as pl
from jax.experimental.pallas import tpu as pltpu

_CHUNK = 5000
_NBUF = 2


def _affine_kernel(x_hbm, w_ref, b_ref, o_hbm, xbuf, obuf, ld_sem, st_sem):
    n = x_hbm.shape[0]
    nc = n // _CHUNK
    w = w_ref[...]
    bias = b_ref[...]

    def ld(i, slot):
        return pltpu.make_async_copy(
            x_hbm.at[pl.ds(i * _CHUNK, _CHUNK), :], xbuf.at[slot],
            ld_sem.at[slot])

    def st(i, slot):
        return pltpu.make_async_copy(
            obuf.at[slot], o_hbm.at[pl.ds(i * _CHUNK, _CHUNK), :],
            st_sem.at[slot])

    for i in range(min(_NBUF, nc)):
        ld(i, i).start()
    for i in range(nc):
        slot = i % _NBUF
        ld(i, slot).wait()
        if i >= _NBUF:
            st(i - _NBUF, slot).wait()
        obuf[slot] = (
            jnp.dot(xbuf[slot], w, preferred_element_type=jnp.float32) + bias
        )
        st(i, slot).start()
        if i + _NBUF < nc:
            ld(i + _NBUF, slot).start()
    for i in range(max(nc - _NBUF, 0), nc):
        st(i, i % _NBUF).wait()


def kernel(nodes, edges, senders, receivers, W, b):
    n, d = nodes.shape
    b2 = b.reshape(1, d)
    return pl.pallas_call(
        _affine_kernel,
        in_specs=[
            pl.BlockSpec(memory_space=pltpu.MemorySpace.HBM),
            pl.BlockSpec(memory_space=pltpu.VMEM),
            pl.BlockSpec(memory_space=pltpu.VMEM),
        ],
        out_specs=pl.BlockSpec(memory_space=pltpu.MemorySpace.HBM),
        out_shape=jax.ShapeDtypeStruct((n, d), jnp.float32),
        scratch_shapes=[
            pltpu.VMEM((_NBUF, _CHUNK, d), jnp.float32),
            pltpu.VMEM((_NBUF, _CHUNK, d), jnp.float32),
            pltpu.SemaphoreType.DMA((_NBUF,)),
            pltpu.SemaphoreType.DMA((_NBUF,)),
        ],
    )(nodes, W, b2)
